# baseline (speedup 1.0000x reference)
import functools

import jax
import jax.numpy as jnp
from jax import lax
from jax.experimental import pallas as pl
from jax.experimental.pallas import tpu as pltpu
from jax.experimental.pallas import tpu_sc as plsc

B, S, D = 4, 2048, 1024
BLOCK_ROWS = 2048

NC, NS, L = 2, 16, 16
NW = NC * NS
RW = S // NW
CHUNK = 16
QPB = RW // CHUNK
NCHUNK = B * QPB


def _add_body(x_ref, e_ref, o_ref):
    o_ref[...] = x_ref[...] + e_ref[...]


def _tc_kernel(x, E):
    grid = (S // BLOCK_ROWS, B)
    return pl.pallas_call(
        _add_body,
        grid=grid,
        in_specs=[
            pl.BlockSpec((1, BLOCK_ROWS, D), lambda i, b: (b, i, 0)),
            pl.BlockSpec((BLOCK_ROWS, D), lambda i, b: (i, 0)),
        ],
        out_specs=pl.BlockSpec((1, BLOCK_ROWS, D), lambda i, b: (b, i, 0)),
        out_shape=jax.ShapeDtypeStruct(x.shape, x.dtype),
    )(x, E)


@functools.cache
def _build_sc_kernel():
    mesh = plsc.VectorSubcoreMesh(core_axis_name="c", subcore_axis_name="s")

    @functools.partial(
        pl.kernel,
        mesh=mesh,
        out_type=jax.ShapeDtypeStruct((B, S, D), jnp.float32),
        scratch_types=[
            pltpu.VMEM((RW, D), jnp.float32),
            pltpu.VMEM((CHUNK, D), jnp.float32),
            pltpu.VMEM((CHUNK, D), jnp.float32),
            pltpu.VMEM((CHUNK, D), jnp.float32),
            pltpu.SemaphoreType.DMA,
            pltpu.SemaphoreType.DMA,
            pltpu.SemaphoreType.DMA,
            pltpu.SemaphoreType.DMA,
            pltpu.SemaphoreType.DMA,
            pltpu.SemaphoreType.DMA,
            pltpu.SemaphoreType.DMA,
        ],
    )
    def sc_add(x_hbm, e_hbm, o_hbm, ebuf, xb0, xb1, xb2,
               esem, is0, is1, is2, os0, os1, os2):
        wid = lax.axis_index("s") * NC + lax.axis_index("c")
        e_base = wid * RW
        xbufs = (xb0, xb1, xb2)
        isems = (is0, is1, is2)
        osems = (os0, os1, os2)

        def src(c):
            b, q = divmod(c, QPB)
            return x_hbm.at[b, pl.ds(e_base + q * CHUNK, CHUNK)]

        def dst(c):
            b, q = divmod(c, QPB)
            return o_hbm.at[b, pl.ds(e_base + q * CHUNK, CHUNK)]

        e_src = e_hbm.at[pl.ds(e_base, RW)]
        pltpu.async_copy(e_src, ebuf, esem)
        pltpu.async_copy(src(0), xbufs[0], isems[0])
        pltpu.make_async_copy(e_src, ebuf, esem).wait()

        for c in range(NCHUNK):
            k = c % 3
            xb = xbufs[k]
            if c >= 2:
                kk = (c - 2) % 3
                pltpu.make_async_copy(xbufs[kk], dst(c - 2), osems[kk]).wait()
            if c + 1 < NCHUNK:
                kn = (c + 1) % 3
                pltpu.async_copy(src(c + 1), xbufs[kn], isems[kn])
            pltpu.make_async_copy(src(c), xb, isems[k]).wait()

            qbase = (c % QPB) * CHUNK
            UNROLL = 8

            @plsc.parallel_loop(0, CHUNK * (D // (L * UNROLL)))
            def _(j):
                r = j // (D // (L * UNROLL))
                cb = (j % (D // (L * UNROLL))) * (L * UNROLL)
                for u in range(UNROLL):
                    sl = pl.ds(cb + u * L, L)
                    plsc.addupdate(xb.at[r, sl], ebuf[qbase + r, sl])

            pltpu.async_copy(xb, dst(c), osems[k])

        for c in range(max(NCHUNK - 2, 0), NCHUNK):
            k = c % 3
            pltpu.make_async_copy(xbufs[k], dst(c), osems[k]).wait()

    return sc_add


def _sc_kernel(x, E):
    return _build_sc_kernel()(x, E)


def kernel(x, E):
    return _sc_kernel(x, E)
